# baseline (speedup 1.0000x reference)
import functools

import jax
import jax.numpy as jnp
from jax import lax
from jax.experimental import pallas as pl
from jax.experimental.pallas import tpu as pltpu
from jax.experimental.pallas import tpu_sc as plsc

EMBED = 576
HEADS = 18
POINTS = 12
FRAMES = 3
DEPTH = 2 * FRAMES - 1
BS = 4
H = 32
W = 32
NQ = H * W
HD = EMBED // HEADS
NW = HD // 2
PAIRS = BS * HEADS
QCHUNKS = 4
QC = NQ // QCHUNKS
TASKS = PAIRS * QCHUNKS
WORKERS = 32
TASKS_PER_W = TASKS // WORKERS
LANES = 16
GROUPS = QC // LANES


def _mm_bias_kernel(a_ref, b_ref, bias_ref, o_ref):
    o_ref[...] = (
        jnp.dot(a_ref[...], b_ref[...], preferred_element_type=jnp.float32)
        + bias_ref[...]
    )


def _mm_bias(a, b, bias, bm, out_dtype=jnp.float32):
    m, k = a.shape
    n = b.shape[1]

    def body(a_ref, b_ref, bias_ref, o_ref):
        acc = jnp.dot(a_ref[...], b_ref[...], preferred_element_type=jnp.float32)
        o_ref[...] = (acc + bias_ref[...]).astype(out_dtype)

    return pl.pallas_call(
        body,
        grid=(m // bm,),
        in_specs=[
            pl.BlockSpec((bm, k), lambda i: (i, 0)),
            pl.BlockSpec((k, n), lambda i: (0, 0)),
            pl.BlockSpec((1, n), lambda i: (0, 0)),
        ],
        out_specs=pl.BlockSpec((bm, n), lambda i: (i, 0)),
        out_shape=jax.ShapeDtypeStruct((m, n), out_dtype),
    )(a, b, bias.reshape(1, n))


def _add_mm_kernel(x_ref, y_ref, b_ref, bias_ref, o_ref):
    q = x_ref[...] + y_ref[...]
    o_ref[...] = (
        jnp.dot(q, b_ref[...], preferred_element_type=jnp.float32)
        + bias_ref[...]
    )


def _add_mm_bias(x, y, b, bias, bm):
    m, k = x.shape
    n = b.shape[1]
    return pl.pallas_call(
        _add_mm_kernel,
        grid=(m // bm,),
        in_specs=[
            pl.BlockSpec((bm, k), lambda i: (i, 0)),
            pl.BlockSpec((bm, k), lambda i: (i, 0)),
            pl.BlockSpec((k, n), lambda i: (0, 0)),
            pl.BlockSpec((1, n), lambda i: (0, 0)),
        ],
        out_specs=pl.BlockSpec((bm, n), lambda i: (i, 0)),
        out_shape=jax.ShapeDtypeStruct((m, n), jnp.float32),
    )(x, y, b, bias.reshape(1, n))


def _clampi(v, hi):
    return jnp.minimum(jnp.maximum(v, 0), hi)


def _floor_parts(coord, hi_f):
    c = jnp.minimum(jnp.maximum(coord, -1.0), hi_f)
    t = c.astype(jnp.int32)
    tf = t.astype(jnp.float32)
    adj = tf > c
    fl_i = t - jnp.where(adj, 1, 0)
    fl_f = tf - jnp.where(adj, 1.0, 0.0)
    return fl_i, c - fl_f


def _sc_sample(vol_words, coords):
    DHW = DEPTH * H * W
    mesh = plsc.VectorSubcoreMesh(core_axis_name="c", subcore_axis_name="s")

    @functools.partial(
        pl.kernel,
        mesh=mesh,
        out_type=jax.ShapeDtypeStruct((PAIRS, QCHUNKS, HD, QC), jnp.float32),
        compiler_params=pltpu.CompilerParams(needs_layout_passes=False),
        scratch_types=[
            pltpu.VMEM((NW * DHW,), jnp.int32),
            pltpu.VMEM((4, POINTS, QC), jnp.float32),
            pltpu.VMEM((POINTS, LANES), jnp.float32),
            pltpu.VMEM((HD, QC), jnp.float32),
        ],
    )
    def sampler(coords_hbm, vol_hbm, out_hbm, vol_v, cv, swv, aggv):
        wid = lax.axis_index("s") * 2 + lax.axis_index("c")
        base_task = wid * TASKS_PER_W

        def task_body(t, prev_pair):
            task = base_task + t
            pair = task // QCHUNKS
            qc = task % QCHUNKS

            @pl.when(pair != prev_pair)
            def _():
                pltpu.sync_copy(vol_hbm.at[pair], vol_v)

            pltpu.sync_copy(coords_hbm.at[pair, qc], cv)

            def g_body(g, carry):
                qo = g * LANES
                logits = [cv[3, p, pl.ds(qo, LANES)] for p in range(POINTS)]
                m = logits[0]
                for p in range(1, POINTS):
                    m = jnp.maximum(m, logits[p])
                exps = [jnp.exp(l - m) for l in logits]
                ssum = exps[0]
                for p in range(1, POINTS):
                    ssum = ssum + exps[p]
                rs = 1.0 / ssum
                for p in range(POINTS):
                    swv[p, :] = exps[p] * rs

                zero = jnp.zeros((LANES,), jnp.float32)
                for ch in range(HD):
                    aggv[ch, pl.ds(qo, LANES)] = zero

                def p_body(p, carry2):
                    zs = cv[2, p, pl.ds(qo, LANES)]
                    zf, fz = _floor_parts(zs, float(DEPTH))
                    zi = (zf, zf + 1)
                    vz = [(d >= 0) & (d <= DEPTH - 1) for d in zi]

                    @pl.when(jnp.any(vz[0] | vz[1]))
                    def _():
                        xs = cv[0, p, pl.ds(qo, LANES)]
                        ys = cv[1, p, pl.ds(qo, LANES)]
                        ws = swv[p, :]
                        xf, fx = _floor_parts(xs, float(W))
                        yf, fy = _floor_parts(ys, float(H))
                        xi = (xf, xf + 1)
                        yi = (yf, yf + 1)
                        vx = [(d >= 0) & (d <= W - 1) for d in xi]
                        vy = [(d >= 0) & (d <= H - 1) for d in yi]

                        @pl.when(jnp.any((vx[0] | vx[1]) & (vy[0] | vy[1])))
                        def _():
                            wx = (1.0 - fx, fx)
                            wy = (1.0 - fy, fy)
                            wz = (1.0 - fz, fz)
                            xc = [_clampi(d, W - 1) for d in xi]
                            ysh = [_clampi(d, H - 1) * W for d in yi]
                            zsh = [_clampi(d, DEPTH - 1) * (H * W)
                                   for d in zi]
                            wzs = [w * ws for w in wz]
                            for dz in (0, 1):
                                for dy in (0, 1):
                                    zy = zsh[dz] + ysh[dy]
                                    vzy = vz[dz] & vy[dy]
                                    wyz = wy[dy] * wzs[dz]
                                    for dx in (0, 1):
                                        row = zy + xc[dx]
                                        wv = jnp.where(vzy & vx[dx],
                                                       wyz * wx[dx], 0.0)
                                        for cw in range(NW):
                                            gw = plsc.load_gather(
                                                vol_v,
                                                [row + jnp.int32(cw * DHW)])
                                            lo = plsc.bitcast(
                                                lax.shift_left(gw, 16),
                                                jnp.float32)
                                            hi = plsc.bitcast(
                                                gw & jnp.int32(-65536),
                                                jnp.float32)
                                            plsc.addupdate(
                                                aggv.at[2 * cw,
                                                        pl.ds(qo, LANES)],
                                                wv * lo)
                                            plsc.addupdate(
                                                aggv.at[2 * cw + 1,
                                                        pl.ds(qo, LANES)],
                                                wv * hi)
                    return carry2

                lax.fori_loop(0, POINTS, p_body, 0)
                return carry

            lax.fori_loop(0, GROUPS, g_body, 0)
            pltpu.sync_copy(aggv, out_hbm.at[pair, qc])
            return pair

        lax.fori_loop(0, TASKS_PER_W, task_body, jnp.int32(-1))

    return sampler(coords, vol_words)


def kernel(query, value, query_pos, spatial_shapes, W_off, b_off, W_attn,
           b_attn, W_val, b_val, W_out, b_out):
    bs, nq, _ = query.shape
    del spatial_shapes

    v = _mm_bias(value.reshape(bs * nq * DEPTH, EMBED), W_val, b_val, 2048,
                 out_dtype=jnp.bfloat16)
    vol = v.reshape(bs, DEPTH * H * W, HEADS, HD).transpose(0, 2, 1, 3)
    vol_words = (
        lax.bitcast_convert_type(
            vol.reshape(PAIRS, DEPTH * H * W, NW, 2), jnp.int32)
        .transpose(0, 2, 1)
        .reshape(PAIRS, NW * DEPTH * H * W)
    )

    W_oa = jnp.concatenate([W_off, W_attn], axis=1)
    b_oa = jnp.concatenate([b_off, b_attn], axis=0)
    oa = _add_mm_bias(query.reshape(bs * nq, EMBED),
                      query_pos.reshape(bs * nq, EMBED), W_oa, b_oa, 1024)
    off = oa[:, :HEADS * POINTS * 3].reshape(bs, nq, HEADS, POINTS, 3)
    attn = oa[:, HEADS * POINTS * 3:].reshape(bs, nq, HEADS, POINTS)

    qs = jnp.arange(nq, dtype=jnp.int32)
    xq = (qs % W).astype(jnp.float32)
    yq = (qs // W).astype(jnp.float32)
    x = off[..., 0] + xq[None, :, None, None]
    y = off[..., 1] + yq[None, :, None, None]
    z = off[..., 2] * (float(DEPTH) / float(FRAMES)) - 0.5
    coords = jnp.stack([x, y, z, attn], axis=0)
    coords = coords.transpose(1, 3, 0, 4, 2)
    coords = coords.reshape(bs, HEADS, 4, POINTS, QCHUNKS, QC)
    coords = coords.transpose(0, 1, 4, 2, 3, 5).reshape(
        PAIRS, QCHUNKS, 4, POINTS, QC)

    agg = _sc_sample(vol_words, coords)

    agg = (agg.reshape(bs, HEADS, QCHUNKS, HD, QC)
           .transpose(0, 2, 4, 1, 3)
           .reshape(bs * nq, EMBED))
    return _mm_bias(agg, W_out, b_out, 1024).reshape(bs, nq, EMBED)


# speedup vs baseline: 1.7188x; 1.7188x over previous
import functools

import jax
import jax.numpy as jnp
from jax import lax
from jax.experimental import pallas as pl
from jax.experimental.pallas import tpu as pltpu
from jax.experimental.pallas import tpu_sc as plsc

EMBED = 576
HEADS = 18
POINTS = 12
FRAMES = 3
DEPTH = 2 * FRAMES - 1
BS = 4
H = 32
W = 32
NQ = H * W
HD = EMBED // HEADS
NW = HD // 2
PAIRS = BS * HEADS
QCHUNKS = 4
QC = NQ // QCHUNKS
TASKS = PAIRS * QCHUNKS
WORKERS = 32
TASKS_PER_W = TASKS // WORKERS
LANES = 16
GROUPS = QC // LANES


def _mm_bias_kernel(a_ref, b_ref, bias_ref, o_ref):
    o_ref[...] = (
        jnp.dot(a_ref[...], b_ref[...], preferred_element_type=jnp.float32)
        + bias_ref[...]
    )


def _mm_bias(a, b, bias, bm, out_dtype=jnp.float32):
    m, k = a.shape
    n = b.shape[1]

    def body(a_ref, b_ref, bias_ref, o_ref):
        acc = jnp.dot(a_ref[...], b_ref[...], preferred_element_type=jnp.float32)
        o_ref[...] = (acc + bias_ref[...]).astype(out_dtype)

    return pl.pallas_call(
        body,
        grid=(m // bm,),
        in_specs=[
            pl.BlockSpec((bm, k), lambda i: (i, 0)),
            pl.BlockSpec((k, n), lambda i: (0, 0)),
            pl.BlockSpec((1, n), lambda i: (0, 0)),
        ],
        out_specs=pl.BlockSpec((bm, n), lambda i: (i, 0)),
        out_shape=jax.ShapeDtypeStruct((m, n), out_dtype),
    )(a, b, bias.reshape(1, n))


def _add_mm_kernel(x_ref, y_ref, b_ref, bias_ref, o_ref):
    q = x_ref[...] + y_ref[...]
    o_ref[...] = (
        jnp.dot(q, b_ref[...], preferred_element_type=jnp.float32)
        + bias_ref[...]
    )


def _add_mm_bias(x, y, b, bias, bm):
    m, k = x.shape
    n = b.shape[1]
    return pl.pallas_call(
        _add_mm_kernel,
        grid=(m // bm,),
        in_specs=[
            pl.BlockSpec((bm, k), lambda i: (i, 0)),
            pl.BlockSpec((bm, k), lambda i: (i, 0)),
            pl.BlockSpec((k, n), lambda i: (0, 0)),
            pl.BlockSpec((1, n), lambda i: (0, 0)),
        ],
        out_specs=pl.BlockSpec((bm, n), lambda i: (i, 0)),
        out_shape=jax.ShapeDtypeStruct((m, n), jnp.float32),
    )(x, y, b, bias.reshape(1, n))


def _clampi(v, hi):
    return jnp.minimum(jnp.maximum(v, 0), hi)


def _floor_parts(coord, hi_f):
    c = jnp.minimum(jnp.maximum(coord, -1.0), hi_f)
    t = c.astype(jnp.int32)
    tf = t.astype(jnp.float32)
    adj = tf > c
    fl_i = t - jnp.where(adj, 1, 0)
    fl_f = tf - jnp.where(adj, 1.0, 0.0)
    return fl_i, c - fl_f


def _sc_sample(vol_words, coords):
    DHW = DEPTH * H * W
    mesh = plsc.VectorSubcoreMesh(core_axis_name="c", subcore_axis_name="s")

    @functools.partial(
        pl.kernel,
        mesh=mesh,
        out_type=jax.ShapeDtypeStruct((PAIRS, QCHUNKS, HD, QC), jnp.float32),
        compiler_params=pltpu.CompilerParams(needs_layout_passes=False),
        scratch_types=[
            pltpu.VMEM((NW * DHW,), jnp.int32),
            pltpu.VMEM((4, POINTS, QC), jnp.float32),
            pltpu.VMEM((POINTS, LANES), jnp.float32),
            pltpu.VMEM((HD, QC), jnp.float32),
        ],
    )
    def sampler(coords_hbm, vol_hbm, out_hbm, vol_v, cv, swv, aggv):
        wid = lax.axis_index("s") * 2 + lax.axis_index("c")
        base_task = wid * TASKS_PER_W

        def task_body(t, prev_pair):
            task = base_task + t
            pair = task // QCHUNKS
            qc = task % QCHUNKS

            @pl.when(pair != prev_pair)
            def _():
                pltpu.sync_copy(vol_hbm.at[pair], vol_v)

            pltpu.sync_copy(coords_hbm.at[pair, qc], cv)

            def g_body(g, carry):
                qo = g * LANES
                logits = [cv[3, p, pl.ds(qo, LANES)] for p in range(POINTS)]
                m = logits[0]
                for p in range(1, POINTS):
                    m = jnp.maximum(m, logits[p])
                exps = [jnp.exp(l - m) for l in logits]
                ssum = exps[0]
                for p in range(1, POINTS):
                    ssum = ssum + exps[p]
                rs = 1.0 / ssum
                for p in range(POINTS):
                    swv[p, :] = exps[p] * rs

                def p_body(p, accs):
                    xs = cv[0, p, pl.ds(qo, LANES)]
                    ys = cv[1, p, pl.ds(qo, LANES)]
                    zs = cv[2, p, pl.ds(qo, LANES)]
                    ws = swv[p, :]
                    xf, fx = _floor_parts(xs, float(W))
                    yf, fy = _floor_parts(ys, float(H))
                    zf, fz = _floor_parts(zs, float(DEPTH))
                    wx = (1.0 - fx, fx)
                    wy = (1.0 - fy, fy)
                    wz = (1.0 - fz, fz)
                    xi = (xf, xf + 1)
                    yi = (yf, yf + 1)
                    zi = (zf, zf + 1)
                    vx = [(d >= 0) & (d <= W - 1) for d in xi]
                    vy = [(d >= 0) & (d <= H - 1) for d in yi]
                    vz = [(d >= 0) & (d <= DEPTH - 1) for d in zi]
                    xc = [_clampi(d, W - 1) for d in xi]
                    ysh = [_clampi(d, H - 1) * W for d in yi]
                    zsh = [_clampi(d, DEPTH - 1) * (H * W) for d in zi]
                    wzs = [w * ws for w in wz]
                    accs = list(accs)
                    for dz in (0, 1):
                        for dy in (0, 1):
                            zy = zsh[dz] + ysh[dy]
                            vzy = vz[dz] & vy[dy]
                            wyz = wy[dy] * wzs[dz]
                            for dx in (0, 1):
                                row = zy + xc[dx]
                                wv = jnp.where(vzy & vx[dx],
                                               wyz * wx[dx], 0.0)
                                for cw in range(NW):
                                    gw = plsc.load_gather(
                                        vol_v, [row + jnp.int32(cw * DHW)])
                                    lo = plsc.bitcast(
                                        lax.shift_left(gw, 16), jnp.float32)
                                    hi = plsc.bitcast(
                                        gw & jnp.int32(-65536), jnp.float32)
                                    accs[2 * cw] = accs[2 * cw] + wv * lo
                                    accs[2 * cw + 1] = (
                                        accs[2 * cw + 1] + wv * hi)
                    return tuple(accs)

                zero = jnp.zeros((LANES,), jnp.float32)
                accs = lax.fori_loop(
                    0, POINTS, p_body, tuple(zero for _ in range(HD)))
                for ch in range(HD):
                    aggv[ch, pl.ds(qo, LANES)] = accs[ch]
                return carry

            lax.fori_loop(0, GROUPS, g_body, 0)
            pltpu.sync_copy(aggv, out_hbm.at[pair, qc])
            return pair

        lax.fori_loop(0, TASKS_PER_W, task_body, jnp.int32(-1))

    return sampler(coords, vol_words)


def kernel(query, value, query_pos, spatial_shapes, W_off, b_off, W_attn,
           b_attn, W_val, b_val, W_out, b_out):
    bs, nq, _ = query.shape
    del spatial_shapes

    v = _mm_bias(value.reshape(bs * nq * DEPTH, EMBED), W_val, b_val, 2048,
                 out_dtype=jnp.bfloat16)
    vol = v.reshape(bs, DEPTH * H * W, HEADS, HD).transpose(0, 2, 1, 3)
    vol_words = (
        lax.bitcast_convert_type(
            vol.reshape(PAIRS, DEPTH * H * W, NW, 2), jnp.int32)
        .transpose(0, 2, 1)
        .reshape(PAIRS, NW * DEPTH * H * W)
    )

    W_oa = jnp.concatenate([W_off, W_attn], axis=1)
    b_oa = jnp.concatenate([b_off, b_attn], axis=0)
    oa = _add_mm_bias(query.reshape(bs * nq, EMBED),
                      query_pos.reshape(bs * nq, EMBED), W_oa, b_oa, 1024)
    off = oa[:, :HEADS * POINTS * 3].reshape(bs, nq, HEADS, POINTS, 3)
    attn = oa[:, HEADS * POINTS * 3:].reshape(bs, nq, HEADS, POINTS)

    qs = jnp.arange(nq, dtype=jnp.int32)
    xq = (qs % W).astype(jnp.float32)
    yq = (qs // W).astype(jnp.float32)
    x = off[..., 0] + xq[None, :, None, None]
    y = off[..., 1] + yq[None, :, None, None]
    z = off[..., 2] * (float(DEPTH) / float(FRAMES)) - 0.5
    coords = jnp.stack([x, y, z, attn], axis=0)
    coords = coords.transpose(1, 3, 0, 4, 2)
    coords = coords.reshape(bs, HEADS, 4, POINTS, QCHUNKS, QC)
    coords = coords.transpose(0, 1, 4, 2, 3, 5).reshape(
        PAIRS, QCHUNKS, 4, POINTS, QC)

    agg = _sc_sample(vol_words, coords)

    agg = (agg.reshape(bs, HEADS, QCHUNKS, HD, QC)
           .transpose(0, 2, 4, 1, 3)
           .reshape(bs * nq, EMBED))
    return _mm_bias(agg, W_out, b_out, 1024).reshape(bs, nq, EMBED)
